# baseline (speedup 1.0000x reference)
import jax
import jax.numpy as jnp
from jax import lax
from jax.experimental import pallas as pl
from jax.experimental.pallas import tpu as pltpu
from jax.experimental.pallas import tpu_sc as plsc

N = 10000
E = 320000
H = 64
NUM_GRAPHS = 16
BE = 3200
EPS = 1e-16

W1 = 128
KCH = 400
SUB = 80
NSUB = KCH // SUB
EPT = E // 16
NCH = EPT // KCH
NHALF = 5000
NPH = 5120
NROW_T = NPH // 16


def _leaky(x, slope=0.2):
    return jnp.where(x >= 0, x, slope * x)


def _elu(x):
    return jnp.where(x > 0, x, jnp.expm1(x))


def _elu_k(x):
    return jnp.where(x > 0, x, jnp.exp(x) - 1.0)


def _pass1_body(src1, dstl, lnp, lep, eattr, h1, zrow,
                out,
                idx0, idx1, idx2, idx3, idx4, lnv, lev, eat, rows, acc, sem):
    c = lax.axis_index("c")
    s = lax.axis_index("s")
    wid = s
    idxs = [idx0, idx1, idx2, idx3, idx4]
    pltpu.sync_copy(zrow.at[pl.ds(0, NROW_T)],
                    acc.at[pl.ds(s * NROW_T, NROW_T)])
    plsc.subcore_barrier()

    def chunk_body(ci, carry):
        base = wid * EPT + ci * KCH
        descs = [
            pltpu.async_copy(lnp.at[pl.ds(base, KCH)], lnv, sem),
            pltpu.async_copy(lep.at[pl.ds(base, KCH)], lev, sem),
            pltpu.async_copy(eattr.at[pl.ds(base * 16, KCH * 16)], eat, sem),
        ] + [
            pltpu.async_copy(src1.at[pl.ds(base + j * SUB, SUB)], idxs[j], sem)
            for j in range(NSUB)
        ]
        for d in descs:
            d.wait()
        descs = [
            pltpu.async_copy(h1.at[idxs[j]],
                             rows.at[pl.ds(j * SUB, SUB)], sem)
            for j in range(NSUB)
        ]
        for d in descs:
            d.wait()

        def group_body(g, gcarry):
            r0 = g * 16
            wn = jnp.exp(_leaky(lnv[pl.ds(r0, 16)]))
            we = jnp.exp(_leaky(lev[pl.ds(r0, 16)]))
            lanes = lax.iota(jnp.int32, 16)
            plsc.store_scatter(rows, [r0 + lanes,
                                      jnp.full((16,), H + 16, jnp.int32)], wn)
            plsc.store_scatter(rows, [r0 + lanes,
                                      jnp.full((16,), H + 17, jnp.int32)], we)
            for k in range(16):
                r = r0 + k
                wnk = wn[k]
                wek = we[k]
                for q in range(H // 16):
                    rows[r, pl.ds(q * 16, 16)] = wnk * rows[r, pl.ds(q * 16, 16)]
                rows[r, pl.ds(H, 16)] = wek * eat[pl.ds(r * 16, 16)]
            return gcarry

        lax.fori_loop(0, KCH // 16, group_body, 0)
        descs = [
            pltpu.async_copy(dstl.at[pl.ds(c * E + base + j * SUB, SUB)],
                             idxs[j], sem)
            for j in range(NSUB)
        ]
        for d in descs:
            d.wait()
        descs = [
            pltpu.async_copy(rows.at[pl.ds(j * SUB, SUB)],
                             acc.at[idxs[j]], sem, add=True)
            for j in range(NSUB)
        ]
        for d in descs:
            d.wait()
        return carry

    lax.fori_loop(0, NCH, chunk_body, 0)
    plsc.subcore_barrier()
    pltpu.sync_copy(acc.at[pl.ds(s * NROW_T, NROW_T)],
                    out.at[c, pl.ds(s * NROW_T, NROW_T)])


def _pass1_call(src, dst, eattr, ln_in, le_in, h1):
    zrow = jnp.zeros((NROW_T, W1), jnp.float32)
    f = pl.kernel(
        _pass1_body,
        out_type=jax.ShapeDtypeStruct((2, NPH, W1), jnp.float32),
        mesh=plsc.VectorSubcoreMesh(core_axis_name="c", subcore_axis_name="s"),
        compiler_params=pltpu.CompilerParams(needs_layout_passes=False),
        scratch_types=[
            pltpu.VMEM((SUB,), jnp.int32),
            pltpu.VMEM((SUB,), jnp.int32),
            pltpu.VMEM((SUB,), jnp.int32),
            pltpu.VMEM((SUB,), jnp.int32),
            pltpu.VMEM((SUB,), jnp.int32),
            pltpu.VMEM((KCH,), jnp.float32),
            pltpu.VMEM((KCH,), jnp.float32),
            pltpu.VMEM((KCH * 16,), jnp.float32),
            pltpu.VMEM((KCH, 128), jnp.float32),
            pltpu.VMEM_SHARED((NPH, W1), jnp.float32),
            pltpu.SemaphoreType.DMA,
        ],
    )
    h1p = jnp.pad(h1, ((0, 0), (0, 128 - H)))
    dl0 = jnp.where(dst < NHALF, dst, NPH - 8)
    dl1 = jnp.where(dst >= NHALF, dst - NHALF, NPH - 8)
    dstl = jnp.concatenate([dl0, dl1])
    parts = f(src, dstl, ln_in, le_in, eattr.reshape(E * 16), h1p, zrow)
    return jnp.concatenate([parts[0, :NHALF], parts[1, :NHALF]], axis=0)


def _edge_pass2_body(eattr_ref, hns_ref, hnd_ref, h2s_ref, h2d_ref, w_ref,
                     out_ref):
    eattr = eattr_ref[...]
    hns = hns_ref[...]
    hnd = hnd_ref[...]
    h2s = h2s_ref[...]
    h2d = h2d_ref[...]
    e1We = w_ref[0:16, 0:H]
    ca = w_ref[16:17, 0:H]
    w1 = w_ref[17:18, 0:H]
    wgb = w_ref[18:19, 0:H]
    a2sv = w_ref[19:20, 0:H]
    wce = w_ref[20:21, 0:16]
    an2d = w_ref[21:22, 0:H]
    ae2d = w_ref[22:23, 0:H]
    c_e = jnp.sum(eattr * wce, axis=1, keepdims=True)
    cns = jnp.sum(hns * ca, axis=1, keepdims=True)
    cnd = jnp.sum(hnd * ca, axis=1, keepdims=True)
    ls = _leaky(c_e + cns)
    ld = _leaky(c_e + cnd)
    m2 = jnp.maximum(ls, ld)
    es = jnp.exp(ls - m2)
    ed = jnp.exp(ld - m2)
    den2 = es + ed + EPS
    enp = _elu_k((es * hns + ed * hnd) / den2)
    g = _elu_k(jnp.dot(eattr, e1We, preferred_element_type=jnp.float32))
    b2 = jnp.sum(enp * w1 + g * wgb, axis=1, keepdims=True)
    a2dd = jnp.sum(h2d * an2d, axis=1, keepdims=True)
    a2ed = jnp.sum(h2d * ae2d, axis=1, keepdims=True)
    a2ss = jnp.sum(h2s * a2sv, axis=1, keepdims=True)
    wn2 = jnp.exp(_leaky(a2dd + a2ss))
    we2 = jnp.exp(_leaky(a2ed + b2))
    zero = jnp.zeros((eattr.shape[0], 256 - 3 * H - 2), jnp.float32)
    out_ref[...] = jnp.concatenate(
        [wn2 * h2s, we2 * enp, we2 * g, wn2, we2, zero], axis=1)


def _seg_sum(v, seg, n):
    return jax.ops.segment_sum(v, seg, num_segments=n)


def kernel(x, edge_attr, edge_index, batch, n1_Wn, n1_an, n1_We, n1_ae, e1_Wn, e1_We, e1_a, n2_Wn, n2_an, n2_We, n2_ae, Wr, br):
    src, dst = edge_index[0], edge_index[1]
    nb = E // BE
    h1 = x @ n1_Wn
    a1d = h1 @ n1_an[:H]
    a1s = h1 @ n1_an[H:]
    a1e = h1 @ n1_ae[:H]
    b1 = edge_attr @ (n1_We @ n1_ae[H:])
    ln_in = a1d[dst] + a1s[src]
    le_in = a1e[dst] + b1

    agg1 = _pass1_call(src, dst, edge_attr, ln_in, le_in, h1)
    den_n = agg1[:, H + 16]
    den_e = agg1[:, H + 17]
    nagg = agg1[:, :H] / (den_n + EPS)[:, None]
    eagg = (agg1[:, H:H + 16] @ n1_We) / (den_e + EPS)[:, None]
    x1 = _elu(jnp.concatenate([nagg, eagg], axis=1))

    hn = x1 @ e1_Wn
    h2 = x1 @ n2_Wn
    ae2v = n2_ae[H:]
    W_np, W_g = n2_We[:H], n2_We[H:]
    w2pack = jnp.zeros((24, 128), jnp.float32)
    w2pack = w2pack.at[0:16, 0:H].set(e1_We)
    w2pack = w2pack.at[16, 0:H].set(e1_a[H:])
    w2pack = w2pack.at[17, 0:H].set(W_np @ ae2v)
    w2pack = w2pack.at[18, 0:H].set(W_g @ ae2v)
    w2pack = w2pack.at[19, 0:H].set(n2_an[H:])
    w2pack = w2pack.at[20, 0:16].set(e1_We @ e1_a[:H])
    w2pack = w2pack.at[21, 0:H].set(n2_an[:H])
    w2pack = w2pack.at[22, 0:H].set(n2_ae[:H])

    espec = pl.BlockSpec((BE, 16), lambda i: (i, 0))
    hspec = pl.BlockSpec((BE, H), lambda i: (i, 0))
    out2 = pl.pallas_call(
        _edge_pass2_body,
        grid=(nb,),
        in_specs=[espec, hspec, hspec, hspec, hspec,
                  pl.BlockSpec((24, 128), lambda i: (0, 0))],
        out_specs=pl.BlockSpec((BE, 256), lambda i: (i, 0)),
        out_shape=jax.ShapeDtypeStruct((E, 256), jnp.float32),
    )(edge_attr, hn[src], hn[dst], h2[src], h2[dst], w2pack)

    agg2 = _seg_sum(out2, dst, N)
    den_n2 = agg2[:, 3 * H]
    den_e2 = agg2[:, 3 * H + 1]
    nagg2 = agg2[:, :H] / (den_n2 + EPS)[:, None]
    eagg2 = (agg2[:, H:2 * H] @ W_np + agg2[:, 2 * H:3 * H] @ W_g) / (
        den_e2 + EPS)[:, None]
    x2 = _elu(jnp.concatenate([nagg2, eagg2], axis=1))

    sums = _seg_sum(x2, batch, NUM_GRAPHS)
    cnts = _seg_sum(jnp.ones((N,), jnp.float32), batch, NUM_GRAPHS)
    gpool = sums / jnp.maximum(cnts, 1.0)[:, None]
    return gpool @ Wr + br


# speedup vs baseline: 2.2807x; 2.2807x over previous
import jax
import jax.numpy as jnp
from jax import lax
from jax.experimental import pallas as pl
from jax.experimental.pallas import tpu as pltpu
from jax.experimental.pallas import tpu_sc as plsc

N = 10000
E = 320000
H = 64
NUM_GRAPHS = 16
BE = 3200
EPS = 1e-16

W1 = 128
KCH = 400
SUB = 80
NSUB = KCH // SUB
EPT = E // 16
NCH = EPT // KCH
NHALF = 5000
NPH = 5120
NROW_T = NPH // 16


def _leaky(x, slope=0.2):
    return jnp.where(x >= 0, x, slope * x)


def _elu(x):
    return jnp.where(x > 0, x, jnp.expm1(x))


def _elu_k(x):
    return jnp.where(x > 0, x, jnp.exp(x) - 1.0)


def _pass1_body(src1, dstl, b1, eattr, h1, a1d, a1s, a1e, zrow,
                out,
                idx0, idx1, idx2, idx3, idx4, srcf, dstf, b1v, eat, rows,
                a1dv, a1sv, a1ev, acc, sem):
    c = lax.axis_index("c")
    s = lax.axis_index("s")
    wid = s
    idxs = [idx0, idx1, idx2, idx3, idx4]
    pltpu.sync_copy(a1d, a1dv)
    pltpu.sync_copy(a1s, a1sv)
    pltpu.sync_copy(a1e, a1ev)
    pltpu.sync_copy(zrow.at[pl.ds(0, NROW_T)],
                    acc.at[pl.ds(s * NROW_T, NROW_T)])
    plsc.subcore_barrier()

    def chunk_body(ci, carry):
        base = wid * EPT + ci * KCH
        descs = [
            pltpu.async_copy(src1.at[pl.ds(base, KCH)], srcf, sem),
            pltpu.async_copy(dstl.at[pl.ds(c * E + base, KCH)], dstf, sem),
            pltpu.async_copy(b1.at[pl.ds(base, KCH)], b1v, sem),
            pltpu.async_copy(eattr.at[pl.ds(base * 16, KCH * 16)], eat, sem),
        ] + [
            pltpu.async_copy(src1.at[pl.ds(base + j * SUB, SUB)], idxs[j], sem)
            for j in range(NSUB)
        ]
        for d in descs:
            d.wait()
        descs = [
            pltpu.async_copy(h1.at[idxs[j]],
                             rows.at[pl.ds(j * SUB, SUB)], sem)
            for j in range(NSUB)
        ]
        for d in descs:
            d.wait()

        def group_body(g, gcarry):
            r0 = g * 16
            srcv = srcf[pl.ds(r0, 16)]
            dstv = dstf[pl.ds(r0, 16)]
            a1dd = plsc.load_gather(a1dv, [dstv])
            a1ss = plsc.load_gather(a1sv, [srcv])
            a1ed = plsc.load_gather(a1ev, [dstv])
            wn = jnp.exp(_leaky(a1dd + a1ss))
            we = jnp.exp(_leaky(a1ed + b1v[pl.ds(r0, 16)]))
            lanes = lax.iota(jnp.int32, 16)
            plsc.store_scatter(rows, [r0 + lanes,
                                      jnp.full((16,), H + 16, jnp.int32)], wn)
            plsc.store_scatter(rows, [r0 + lanes,
                                      jnp.full((16,), H + 17, jnp.int32)], we)
            for k in range(16):
                r = r0 + k
                wnk = wn[k]
                wek = we[k]
                for q in range(H // 16):
                    rows[r, pl.ds(q * 16, 16)] = wnk * rows[r, pl.ds(q * 16, 16)]
                rows[r, pl.ds(H, 16)] = wek * eat[pl.ds(r * 16, 16)]
            return gcarry

        lax.fori_loop(0, KCH // 16, group_body, 0)
        descs = [
            pltpu.async_copy(dstl.at[pl.ds(c * E + base + j * SUB, SUB)],
                             idxs[j], sem)
            for j in range(NSUB)
        ]
        for d in descs:
            d.wait()
        descs = [
            pltpu.async_copy(rows.at[pl.ds(j * SUB, SUB)],
                             acc.at[idxs[j]], sem, add=True)
            for j in range(NSUB)
        ]
        for d in descs:
            d.wait()
        return carry

    lax.fori_loop(0, NCH, chunk_body, 0)
    plsc.subcore_barrier()
    pltpu.sync_copy(acc.at[pl.ds(s * NROW_T, NROW_T)],
                    out.at[c, pl.ds(s * NROW_T, NROW_T)])


def _pass1_call(src, dst, eattr, b1, h1, a1d, a1s, a1e):
    zrow = jnp.zeros((NROW_T, W1), jnp.float32)
    f = pl.kernel(
        _pass1_body,
        out_type=jax.ShapeDtypeStruct((2, NPH, W1), jnp.float32),
        mesh=plsc.VectorSubcoreMesh(core_axis_name="c", subcore_axis_name="s"),
        compiler_params=pltpu.CompilerParams(needs_layout_passes=False),
        scratch_types=[
            pltpu.VMEM((SUB,), jnp.int32),
            pltpu.VMEM((SUB,), jnp.int32),
            pltpu.VMEM((SUB,), jnp.int32),
            pltpu.VMEM((SUB,), jnp.int32),
            pltpu.VMEM((SUB,), jnp.int32),
            pltpu.VMEM((KCH,), jnp.int32),
            pltpu.VMEM((KCH,), jnp.int32),
            pltpu.VMEM((KCH,), jnp.float32),
            pltpu.VMEM((KCH * 16,), jnp.float32),
            pltpu.VMEM((KCH, 128), jnp.float32),
            pltpu.VMEM((N,), jnp.float32),
            pltpu.VMEM((N,), jnp.float32),
            pltpu.VMEM((N,), jnp.float32),
            pltpu.VMEM_SHARED((NPH, W1), jnp.float32),
            pltpu.SemaphoreType.DMA,
        ],
    )
    h1p = jnp.pad(h1, ((0, 0), (0, 128 - H)))
    dl0 = jnp.where(dst < NHALF, dst, NPH - 8)
    dl1 = jnp.where(dst >= NHALF, dst - NHALF, NPH - 8)
    dstl = jnp.concatenate([dl0, dl1])
    parts = f(src, dstl, b1, eattr.reshape(E * 16), h1p, a1d, a1s, a1e, zrow)
    return jnp.concatenate([parts[0, :NHALF], parts[1, :NHALF]], axis=0)


def _edge_pass2_body(eattr_ref, hns_ref, hnd_ref, h2s_ref, h2d_ref, w_ref,
                     out_ref):
    eattr = eattr_ref[...]
    hns = hns_ref[...]
    hnd = hnd_ref[...]
    h2s = h2s_ref[...]
    h2d = h2d_ref[...]
    e1We = w_ref[0:16, 0:H]
    ca = w_ref[16:17, 0:H]
    w1 = w_ref[17:18, 0:H]
    wgb = w_ref[18:19, 0:H]
    a2sv = w_ref[19:20, 0:H]
    wce = w_ref[20:21, 0:16]
    an2d = w_ref[21:22, 0:H]
    ae2d = w_ref[22:23, 0:H]
    c_e = jnp.sum(eattr * wce, axis=1, keepdims=True)
    cns = jnp.sum(hns * ca, axis=1, keepdims=True)
    cnd = jnp.sum(hnd * ca, axis=1, keepdims=True)
    ls = _leaky(c_e + cns)
    ld = _leaky(c_e + cnd)
    m2 = jnp.maximum(ls, ld)
    es = jnp.exp(ls - m2)
    ed = jnp.exp(ld - m2)
    den2 = es + ed + EPS
    enp = _elu_k((es * hns + ed * hnd) / den2)
    g = _elu_k(jnp.dot(eattr, e1We, preferred_element_type=jnp.float32))
    b2 = jnp.sum(enp * w1 + g * wgb, axis=1, keepdims=True)
    a2dd = jnp.sum(h2d * an2d, axis=1, keepdims=True)
    a2ed = jnp.sum(h2d * ae2d, axis=1, keepdims=True)
    a2ss = jnp.sum(h2s * a2sv, axis=1, keepdims=True)
    wn2 = jnp.exp(_leaky(a2dd + a2ss))
    we2 = jnp.exp(_leaky(a2ed + b2))
    zero = jnp.zeros((eattr.shape[0], 256 - 3 * H - 2), jnp.float32)
    out_ref[...] = jnp.concatenate(
        [wn2 * h2s, we2 * enp, we2 * g, wn2, we2, zero], axis=1)


def _seg_sum(v, seg, n):
    return jax.ops.segment_sum(v, seg, num_segments=n)


def kernel(x, edge_attr, edge_index, batch, n1_Wn, n1_an, n1_We, n1_ae, e1_Wn, e1_We, e1_a, n2_Wn, n2_an, n2_We, n2_ae, Wr, br):
    src, dst = edge_index[0], edge_index[1]
    nb = E // BE
    h1 = x @ n1_Wn
    a1d = h1 @ n1_an[:H]
    a1s = h1 @ n1_an[H:]
    a1e = h1 @ n1_ae[:H]
    b1 = edge_attr @ (n1_We @ n1_ae[H:])

    agg1 = _pass1_call(src, dst, edge_attr, b1, h1, a1d, a1s, a1e)
    den_n = agg1[:, H + 16]
    den_e = agg1[:, H + 17]
    nagg = agg1[:, :H] / (den_n + EPS)[:, None]
    eagg = (agg1[:, H:H + 16] @ n1_We) / (den_e + EPS)[:, None]
    x1 = _elu(jnp.concatenate([nagg, eagg], axis=1))

    hn = x1 @ e1_Wn
    h2 = x1 @ n2_Wn
    ae2v = n2_ae[H:]
    W_np, W_g = n2_We[:H], n2_We[H:]
    w2pack = jnp.zeros((24, 128), jnp.float32)
    w2pack = w2pack.at[0:16, 0:H].set(e1_We)
    w2pack = w2pack.at[16, 0:H].set(e1_a[H:])
    w2pack = w2pack.at[17, 0:H].set(W_np @ ae2v)
    w2pack = w2pack.at[18, 0:H].set(W_g @ ae2v)
    w2pack = w2pack.at[19, 0:H].set(n2_an[H:])
    w2pack = w2pack.at[20, 0:16].set(e1_We @ e1_a[:H])
    w2pack = w2pack.at[21, 0:H].set(n2_an[:H])
    w2pack = w2pack.at[22, 0:H].set(n2_ae[:H])

    espec = pl.BlockSpec((BE, 16), lambda i: (i, 0))
    hspec = pl.BlockSpec((BE, H), lambda i: (i, 0))
    out2 = pl.pallas_call(
        _edge_pass2_body,
        grid=(nb,),
        in_specs=[espec, hspec, hspec, hspec, hspec,
                  pl.BlockSpec((24, 128), lambda i: (0, 0))],
        out_specs=pl.BlockSpec((BE, 256), lambda i: (i, 0)),
        out_shape=jax.ShapeDtypeStruct((E, 256), jnp.float32),
    )(edge_attr, hn[src], hn[dst], h2[src], h2[dst], w2pack)

    agg2 = _seg_sum(out2, dst, N)
    den_n2 = agg2[:, 3 * H]
    den_e2 = agg2[:, 3 * H + 1]
    nagg2 = agg2[:, :H] / (den_n2 + EPS)[:, None]
    eagg2 = (agg2[:, H:2 * H] @ W_np + agg2[:, 2 * H:3 * H] @ W_g) / (
        den_e2 + EPS)[:, None]
    x2 = _elu(jnp.concatenate([nagg2, eagg2], axis=1))

    sums = _seg_sum(x2, batch, NUM_GRAPHS)
    cnts = _seg_sum(jnp.ones((N,), jnp.float32), batch, NUM_GRAPHS)
    gpool = sums / jnp.maximum(cnts, 1.0)[:, None]
    return gpool @ Wr + br
